# Initial kernel scaffold; baseline (speedup 1.0000x reference)
import jax
import jax.numpy as jnp
from jax.experimental import pallas as pl


def kernel(x, edge_index, W_l, b_l, W_r):
    raise NotImplementedError("write your pallas kernel here")


# speedup vs baseline: 4.7478x; 4.7478x over previous
import functools

import jax
import jax.numpy as jnp
from jax import lax
from jax.experimental import pallas as pl
from jax.experimental.pallas import tpu as pltpu
from jax.experimental.pallas import tpu_sc as plsc

NC = 2
NS = 16
L = 16

CHUNK = 80


def _own_blocks(s, NBLK, ROUNDS, do_copies):
    def body(rnd, carry):
        blk = rnd * NS + s

        @pl.when(blk < NBLK)
        def _():
            do_copies(blk * CHUNK)
        return carry

    lax.fori_loop(0, ROUNDS, body, 0)


def _sc_segment_sum(N, E, D, x, src, dst):
    EPW = E // (NC * NS)
    NCHUNK = EPW // CHUNK
    NBLK = N // CHUNK
    ROUNDS = (NBLK + NS - 1) // NS

    mesh = plsc.VectorSubcoreMesh(core_axis_name="c", subcore_axis_name="s")

    @functools.partial(
        pl.kernel,
        out_type=jax.ShapeDtypeStruct((NC, N, D), jnp.float32),
        mesh=mesh,
        scratch_types=[
            pltpu.VMEM((CHUNK,), jnp.int32),
            pltpu.VMEM((CHUNK,), jnp.int32),
            pltpu.VMEM((CHUNK, D), jnp.float32),
            pltpu.VMEM_SHARED((N, D), jnp.float32),
            pltpu.SemaphoreType.DMA,
        ],
    )
    def seg_sum(x_hbm, src_hbm, dst_hbm, psum_hbm, src_v, dst_v, rows_v, acc_sh, sem):
        c = lax.axis_index("c")
        s = lax.axis_index("s")
        zeros16 = jnp.zeros((L,), jnp.float32)

        def zrow(r, carry):
            for j in range(D // L):
                rows_v[r, pl.ds(j * L, L)] = zeros16
            return carry

        lax.fori_loop(0, CHUNK, zrow, 0)

        _own_blocks(s, NBLK, ROUNDS,
                    lambda r0: pltpu.sync_copy(rows_v, acc_sh.at[pl.ds(r0, CHUNK)]))

        plsc.subcore_barrier()

        base = (c * NS + s) * EPW

        def chunk_fn(i, carry):
            off = base + i * CHUNK
            pltpu.sync_copy(src_hbm.at[pl.ds(off, CHUNK)], src_v)
            pltpu.sync_copy(dst_hbm.at[pl.ds(off, CHUNK)], dst_v)
            pltpu.async_copy(x_hbm.at[src_v], rows_v, sem).wait()
            pltpu.sync_copy(rows_v, acc_sh.at[dst_v], add=True)
            return carry

        lax.fori_loop(0, NCHUNK, chunk_fn, 0)

        plsc.subcore_barrier()

        _own_blocks(s, NBLK, ROUNDS,
                    lambda r0: pltpu.sync_copy(acc_sh.at[pl.ds(r0, CHUNK)],
                                               psum_hbm.at[c, pl.ds(r0, CHUNK)]))

    return seg_sum(x, src, dst)


def _sc_segment_count(N, E, D, dst):
    EPW = E // (NC * NS)
    NCHUNK = EPW // CHUNK
    NBLK = N // CHUNK
    ROUNDS = (NBLK + NS - 1) // NS

    mesh = plsc.VectorSubcoreMesh(core_axis_name="c", subcore_axis_name="s")

    @functools.partial(
        pl.kernel,
        out_type=jax.ShapeDtypeStruct((NC, N, D), jnp.float32),
        mesh=mesh,
        scratch_types=[
            pltpu.VMEM((CHUNK,), jnp.int32),
            pltpu.VMEM((CHUNK, D), jnp.float32),
            pltpu.VMEM((CHUNK, D), jnp.float32),
            pltpu.VMEM_SHARED((N, D), jnp.float32),
        ],
    )
    def seg_cnt(dst_hbm, pcnt_hbm, dst_v, ones_v, zb_v, cnt_sh):
        c = lax.axis_index("c")
        s = lax.axis_index("s")
        zeros16 = jnp.zeros((L,), jnp.float32)
        ones16 = jnp.ones((L,), jnp.float32)

        def frow(r, carry):
            for j in range(D // L):
                ones_v[r, pl.ds(j * L, L)] = ones16
                zb_v[r, pl.ds(j * L, L)] = zeros16
            return carry

        lax.fori_loop(0, CHUNK, frow, 0)

        _own_blocks(s, NBLK, ROUNDS,
                    lambda r0: pltpu.sync_copy(zb_v, cnt_sh.at[pl.ds(r0, CHUNK)]))

        plsc.subcore_barrier()

        base = (c * NS + s) * EPW

        def chunk_fn(i, carry):
            off = base + i * CHUNK
            pltpu.sync_copy(dst_hbm.at[pl.ds(off, CHUNK)], dst_v)
            pltpu.sync_copy(ones_v, cnt_sh.at[dst_v], add=True)
            return carry

        lax.fori_loop(0, NCHUNK, chunk_fn, 0)

        plsc.subcore_barrier()

        _own_blocks(s, NBLK, ROUNDS,
                    lambda r0: pltpu.sync_copy(cnt_sh.at[pl.ds(r0, CHUNK)],
                                               pcnt_hbm.at[c, pl.ds(r0, CHUNK)]))

    return seg_cnt(dst)


def _tc_combine(N, D, H, BLK, p0, p1, c0, c1, x, W_l, W_r, b_l):
    def body(p0_ref, p1_ref, c0_ref, c1_ref, x_ref, wl_ref, wr_ref, b_ref, o_ref):
        ssum = p0_ref[...] + p1_ref[...]
        cnt = c0_ref[...] + c1_ref[...]
        denom = jnp.maximum(cnt[:, 0:1], 1.0)
        mean = ssum / denom
        o_ref[...] = (
            jnp.dot(mean, wl_ref[...], preferred_element_type=jnp.float32)
            + jnp.dot(x_ref[...], wr_ref[...], preferred_element_type=jnp.float32)
            + b_ref[...]
        )

    grid = (N // BLK,)
    return pl.pallas_call(
        body,
        grid=grid,
        in_specs=[
            pl.BlockSpec((BLK, D), lambda i: (i, 0)),
            pl.BlockSpec((BLK, D), lambda i: (i, 0)),
            pl.BlockSpec((BLK, D), lambda i: (i, 0)),
            pl.BlockSpec((BLK, D), lambda i: (i, 0)),
            pl.BlockSpec((BLK, D), lambda i: (i, 0)),
            pl.BlockSpec((D, H), lambda i: (0, 0)),
            pl.BlockSpec((D, H), lambda i: (0, 0)),
            pl.BlockSpec((1, H), lambda i: (0, 0)),
        ],
        out_specs=pl.BlockSpec((BLK, H), lambda i: (i, 0)),
        out_shape=jax.ShapeDtypeStruct((N, H), jnp.float32),
    )(p0, p1, c0, c1, x, W_l, W_r, b_l)


def kernel(x, edge_index, W_l, b_l, W_r):
    N, D = x.shape
    E = edge_index.shape[1]
    H = W_l.shape[1]

    src = edge_index[0]
    dst = edge_index[1]

    psum = _sc_segment_sum(N, E, D, x, src, dst)
    pcnt = _sc_segment_count(N, E, D, dst)
    out = _tc_combine(N, D, H, 1000, psum[0], psum[1], pcnt[0], pcnt[1],
                      x, W_l, W_r, b_l.reshape(1, H))
    return out


# speedup vs baseline: 7.1761x; 1.5114x over previous
import functools

import jax
import jax.numpy as jnp
from jax import lax
from jax.experimental import pallas as pl
from jax.experimental.pallas import tpu as pltpu
from jax.experimental.pallas import tpu_sc as plsc

NC = 2
NS = 16
L = 16

CHUNK = 80
NBUF = 4


def _own_blocks(s, NBLK, ROUNDS, do_copies):
    def body(rnd, carry):
        blk = rnd * NS + s

        @pl.when(blk < NBLK)
        def _():
            do_copies(blk * CHUNK)
        return carry

    lax.fori_loop(0, ROUNDS, body, 0)


def _sc_segment_sum(N, E, D, x, src, dst):
    EPW = E // (NC * NS)
    NCHUNK = EPW // CHUNK
    NBLK = N // CHUNK
    ROUNDS = (NBLK + NS - 1) // NS

    mesh = plsc.VectorSubcoreMesh(core_axis_name="c", subcore_axis_name="s")

    @functools.partial(
        pl.kernel,
        out_type=jax.ShapeDtypeStruct((NC, N, D), jnp.float32),
        mesh=mesh,
        scratch_types=[
            [pltpu.VMEM((CHUNK,), jnp.int32)] * NBUF,
            [pltpu.VMEM((CHUNK,), jnp.int32)] * NBUF,
            [pltpu.VMEM((CHUNK, D), jnp.float32)] * NBUF,
            pltpu.VMEM_SHARED((N, D), jnp.float32),
            [pltpu.SemaphoreType.DMA] * NBUF,
        ],
    )
    def seg_sum(x_hbm, src_hbm, dst_hbm, psum_hbm, src_vs, dst_vs, rows_vs, acc_sh, sems):
        c = lax.axis_index("c")
        s = lax.axis_index("s")
        zeros16 = jnp.zeros((L,), jnp.float32)

        def zrow(r, carry):
            for j in range(D // L):
                rows_vs[0][r, pl.ds(j * L, L)] = zeros16
            return carry

        lax.fori_loop(0, CHUNK, zrow, 0)

        _own_blocks(s, NBLK, ROUNDS,
                    lambda r0: pltpu.sync_copy(rows_vs[0], acc_sh.at[pl.ds(r0, CHUNK)]))

        plsc.subcore_barrier()

        base = (c * NS + s) * EPW
        NGRP = NCHUNK // NBUF

        def grp_fn(g, carry):
            offg = base + g * (NBUF * CHUNK)
            descs = []
            for b in range(NBUF):
                pltpu.sync_copy(src_hbm.at[pl.ds(offg + b * CHUNK, CHUNK)], src_vs[b])
                descs.append(pltpu.async_copy(x_hbm.at[src_vs[b]], rows_vs[b], sems[b]))
            for b in range(NBUF):
                pltpu.sync_copy(dst_hbm.at[pl.ds(offg + b * CHUNK, CHUNK)], dst_vs[b])
                descs[b].wait()
                pltpu.sync_copy(rows_vs[b], acc_sh.at[dst_vs[b]], add=True)
            return carry

        lax.fori_loop(0, NGRP, grp_fn, 0)

        for i in range(NGRP * NBUF, NCHUNK):
            off = base + i * CHUNK
            pltpu.sync_copy(src_hbm.at[pl.ds(off, CHUNK)], src_vs[0])
            pltpu.sync_copy(dst_hbm.at[pl.ds(off, CHUNK)], dst_vs[0])
            pltpu.async_copy(x_hbm.at[src_vs[0]], rows_vs[0], sems[0]).wait()
            pltpu.sync_copy(rows_vs[0], acc_sh.at[dst_vs[0]], add=True)

        plsc.subcore_barrier()

        _own_blocks(s, NBLK, ROUNDS,
                    lambda r0: pltpu.sync_copy(acc_sh.at[pl.ds(r0, CHUNK)],
                                               psum_hbm.at[c, pl.ds(r0, CHUNK)]))

    return seg_sum(x, src, dst)


def _sc_segment_count(N, E, D, dst):
    EPW = E // (NC * NS)
    NCHUNK = EPW // CHUNK
    NBLK = N // CHUNK
    ROUNDS = (NBLK + NS - 1) // NS

    mesh = plsc.VectorSubcoreMesh(core_axis_name="c", subcore_axis_name="s")

    @functools.partial(
        pl.kernel,
        out_type=jax.ShapeDtypeStruct((NC, N, D), jnp.float32),
        mesh=mesh,
        scratch_types=[
            [pltpu.VMEM((CHUNK,), jnp.int32)] * NBUF,
            pltpu.VMEM((CHUNK, D), jnp.float32),
            pltpu.VMEM((CHUNK, D), jnp.float32),
            pltpu.VMEM_SHARED((N, D), jnp.float32),
            [pltpu.SemaphoreType.DMA] * NBUF,
        ],
    )
    def seg_cnt(dst_hbm, pcnt_hbm, dst_vs, ones_v, zb_v, cnt_sh, sems):
        c = lax.axis_index("c")
        s = lax.axis_index("s")
        zeros16 = jnp.zeros((L,), jnp.float32)
        ones16 = jnp.ones((L,), jnp.float32)

        def frow(r, carry):
            for j in range(D // L):
                ones_v[r, pl.ds(j * L, L)] = ones16
                zb_v[r, pl.ds(j * L, L)] = zeros16
            return carry

        lax.fori_loop(0, CHUNK, frow, 0)

        _own_blocks(s, NBLK, ROUNDS,
                    lambda r0: pltpu.sync_copy(zb_v, cnt_sh.at[pl.ds(r0, CHUNK)]))

        plsc.subcore_barrier()

        base = (c * NS + s) * EPW
        NGRP = NCHUNK // NBUF

        def grp_fn(g, carry):
            offg = base + g * (NBUF * CHUNK)
            descs = []
            for b in range(NBUF):
                pltpu.sync_copy(dst_hbm.at[pl.ds(offg + b * CHUNK, CHUNK)], dst_vs[b])
                descs.append(pltpu.async_copy(ones_v, cnt_sh.at[dst_vs[b]], sems[b], add=True))
            for b in range(NBUF):
                descs[b].wait()
            return carry

        lax.fori_loop(0, NGRP, grp_fn, 0)

        for i in range(NGRP * NBUF, NCHUNK):
            off = base + i * CHUNK
            pltpu.sync_copy(dst_hbm.at[pl.ds(off, CHUNK)], dst_vs[0])
            pltpu.sync_copy(ones_v, cnt_sh.at[dst_vs[0]], add=True)

        plsc.subcore_barrier()

        _own_blocks(s, NBLK, ROUNDS,
                    lambda r0: pltpu.sync_copy(cnt_sh.at[pl.ds(r0, CHUNK)],
                                               pcnt_hbm.at[c, pl.ds(r0, CHUNK)]))

    return seg_cnt(dst)


def _tc_combine(N, D, H, BLK, p0, p1, c0, c1, x, W_l, W_r, b_l):
    def body(p0_ref, p1_ref, c0_ref, c1_ref, x_ref, wl_ref, wr_ref, b_ref, o_ref):
        ssum = p0_ref[...] + p1_ref[...]
        cnt = c0_ref[...] + c1_ref[...]
        denom = jnp.maximum(cnt[:, 0:1], 1.0)
        mean = ssum / denom
        o_ref[...] = (
            jnp.dot(mean, wl_ref[...], preferred_element_type=jnp.float32)
            + jnp.dot(x_ref[...], wr_ref[...], preferred_element_type=jnp.float32)
            + b_ref[...]
        )

    grid = (N // BLK,)
    return pl.pallas_call(
        body,
        grid=grid,
        in_specs=[
            pl.BlockSpec((BLK, D), lambda i: (i, 0)),
            pl.BlockSpec((BLK, D), lambda i: (i, 0)),
            pl.BlockSpec((BLK, D), lambda i: (i, 0)),
            pl.BlockSpec((BLK, D), lambda i: (i, 0)),
            pl.BlockSpec((BLK, D), lambda i: (i, 0)),
            pl.BlockSpec((D, H), lambda i: (0, 0)),
            pl.BlockSpec((D, H), lambda i: (0, 0)),
            pl.BlockSpec((1, H), lambda i: (0, 0)),
        ],
        out_specs=pl.BlockSpec((BLK, H), lambda i: (i, 0)),
        out_shape=jax.ShapeDtypeStruct((N, H), jnp.float32),
    )(p0, p1, c0, c1, x, W_l, W_r, b_l)


def kernel(x, edge_index, W_l, b_l, W_r):
    N, D = x.shape
    E = edge_index.shape[1]
    H = W_l.shape[1]

    src = edge_index[0]
    dst = edge_index[1]

    psum = _sc_segment_sum(N, E, D, x, src, dst)
    pcnt = _sc_segment_count(N, E, D, dst)
    out = _tc_combine(N, D, H, 1000, psum[0], psum[1], pcnt[0], pcnt[1],
                      x, W_l, W_r, b_l.reshape(1, H))
    return out


# speedup vs baseline: 7.2545x; 1.0109x over previous
import functools

import jax
import jax.numpy as jnp
from jax import lax
from jax.experimental import pallas as pl
from jax.experimental.pallas import tpu as pltpu
from jax.experimental.pallas import tpu_sc as plsc

NC = 2
NS = 16
L = 16

CHUNK = 80
NBUF = 4


def _own_blocks(s, NBLK, ROUNDS, do_copies):
    def body(rnd, carry):
        blk = rnd * NS + s

        @pl.when(blk < NBLK)
        def _():
            do_copies(blk * CHUNK)
        return carry

    lax.fori_loop(0, ROUNDS, body, 0)


def _sc_segment_sum(N, E, D, x, src, dst):
    EPW = E // (NC * NS)
    NCHUNK = EPW // CHUNK
    NBLK = N // CHUNK
    ROUNDS = (NBLK + NS - 1) // NS

    mesh = plsc.VectorSubcoreMesh(core_axis_name="c", subcore_axis_name="s")

    @functools.partial(
        pl.kernel,
        out_type=jax.ShapeDtypeStruct((NC, N, D), jnp.float32),
        mesh=mesh,
        scratch_types=[
            [pltpu.VMEM((CHUNK,), jnp.int32)] * NBUF,
            [pltpu.VMEM((CHUNK,), jnp.int32)] * NBUF,
            [pltpu.VMEM((CHUNK, D), jnp.float32)] * NBUF,
            pltpu.VMEM_SHARED((N, D), jnp.float32),
            [pltpu.SemaphoreType.DMA] * NBUF,
        ],
    )
    def seg_sum(x_hbm, src_hbm, dst_hbm, psum_hbm, src_vs, dst_vs, rows_vs, acc_sh, sems):
        c = lax.axis_index("c")
        s = lax.axis_index("s")
        zeros16 = jnp.zeros((L,), jnp.float32)

        def zrow(r, carry):
            for j in range(D // L):
                rows_vs[0][r, pl.ds(j * L, L)] = zeros16
            return carry

        lax.fori_loop(0, CHUNK, zrow, 0)

        _own_blocks(s, NBLK, ROUNDS,
                    lambda r0: pltpu.sync_copy(rows_vs[0], acc_sh.at[pl.ds(r0, CHUNK)]))

        plsc.subcore_barrier()

        base = (c * NS + s) * EPW
        NGRP = NCHUNK // NBUF

        for b in range(NBUF):
            pltpu.sync_copy(src_hbm.at[pl.ds(base + b * CHUNK, CHUNK)], src_vs[b])
            pltpu.async_copy(x_hbm.at[src_vs[b]], rows_vs[b], sems[b])

        def grp_fn(g, carry):
            for b in range(NBUF):
                i = g * NBUF + b
                off = base + i * CHUNK
                pltpu.sync_copy(dst_hbm.at[pl.ds(off, CHUNK)], dst_vs[b])
                pltpu.make_async_copy(x_hbm.at[src_vs[b]], rows_vs[b], sems[b]).wait()
                pltpu.sync_copy(rows_vs[b], acc_sh.at[dst_vs[b]], add=True)

                @pl.when(i + NBUF < NCHUNK)
                def _():
                    offn = base + (i + NBUF) * CHUNK
                    pltpu.sync_copy(src_hbm.at[pl.ds(offn, CHUNK)], src_vs[b])
                    pltpu.async_copy(x_hbm.at[src_vs[b]], rows_vs[b], sems[b])
            return carry

        lax.fori_loop(0, NGRP, grp_fn, 0)

        for i in range(NGRP * NBUF, NCHUNK):
            b = i % NBUF
            off = base + i * CHUNK
            pltpu.sync_copy(dst_hbm.at[pl.ds(off, CHUNK)], dst_vs[b])
            pltpu.make_async_copy(x_hbm.at[src_vs[b]], rows_vs[b], sems[b]).wait()
            pltpu.sync_copy(rows_vs[b], acc_sh.at[dst_vs[b]], add=True)

        plsc.subcore_barrier()

        _own_blocks(s, NBLK, ROUNDS,
                    lambda r0: pltpu.sync_copy(acc_sh.at[pl.ds(r0, CHUNK)],
                                               psum_hbm.at[c, pl.ds(r0, CHUNK)]))

    return seg_sum(x, src, dst)


def _sc_segment_count(N, E, D, dst):
    EPW = E // (NC * NS)
    NCHUNK = EPW // CHUNK
    NBLK = N // CHUNK
    ROUNDS = (NBLK + NS - 1) // NS

    mesh = plsc.VectorSubcoreMesh(core_axis_name="c", subcore_axis_name="s")

    @functools.partial(
        pl.kernel,
        out_type=jax.ShapeDtypeStruct((NC, N, D), jnp.float32),
        mesh=mesh,
        scratch_types=[
            [pltpu.VMEM((CHUNK,), jnp.int32)] * NBUF,
            pltpu.VMEM((CHUNK, D), jnp.float32),
            pltpu.VMEM((CHUNK, D), jnp.float32),
            pltpu.VMEM_SHARED((N, D), jnp.float32),
            [pltpu.SemaphoreType.DMA] * NBUF,
        ],
    )
    def seg_cnt(dst_hbm, pcnt_hbm, dst_vs, ones_v, zb_v, cnt_sh, sems):
        c = lax.axis_index("c")
        s = lax.axis_index("s")
        zeros16 = jnp.zeros((L,), jnp.float32)
        ones16 = jnp.ones((L,), jnp.float32)

        def frow(r, carry):
            for j in range(D // L):
                ones_v[r, pl.ds(j * L, L)] = ones16
                zb_v[r, pl.ds(j * L, L)] = zeros16
            return carry

        lax.fori_loop(0, CHUNK, frow, 0)

        _own_blocks(s, NBLK, ROUNDS,
                    lambda r0: pltpu.sync_copy(zb_v, cnt_sh.at[pl.ds(r0, CHUNK)]))

        plsc.subcore_barrier()

        base = (c * NS + s) * EPW
        NGRP = NCHUNK // NBUF

        def grp_fn(g, carry):
            offg = base + g * (NBUF * CHUNK)
            descs = []
            for b in range(NBUF):
                pltpu.sync_copy(dst_hbm.at[pl.ds(offg + b * CHUNK, CHUNK)], dst_vs[b])
                descs.append(pltpu.async_copy(ones_v, cnt_sh.at[dst_vs[b]], sems[b], add=True))
            for b in range(NBUF):
                descs[b].wait()
            return carry

        lax.fori_loop(0, NGRP, grp_fn, 0)

        for i in range(NGRP * NBUF, NCHUNK):
            off = base + i * CHUNK
            pltpu.sync_copy(dst_hbm.at[pl.ds(off, CHUNK)], dst_vs[0])
            pltpu.sync_copy(ones_v, cnt_sh.at[dst_vs[0]], add=True)

        plsc.subcore_barrier()

        _own_blocks(s, NBLK, ROUNDS,
                    lambda r0: pltpu.sync_copy(cnt_sh.at[pl.ds(r0, CHUNK)],
                                               pcnt_hbm.at[c, pl.ds(r0, CHUNK)]))

    return seg_cnt(dst)


def _tc_self(N, D, H, BLK, x, W_r, b_l):
    def body(x_ref, wr_ref, b_ref, o_ref):
        o_ref[...] = (
            jnp.dot(x_ref[...], wr_ref[...], preferred_element_type=jnp.float32)
            + b_ref[...]
        )

    return pl.pallas_call(
        body,
        grid=(N // BLK,),
        in_specs=[
            pl.BlockSpec((BLK, D), lambda i: (i, 0)),
            pl.BlockSpec((D, H), lambda i: (0, 0)),
            pl.BlockSpec((1, H), lambda i: (0, 0)),
        ],
        out_specs=pl.BlockSpec((BLK, H), lambda i: (i, 0)),
        out_shape=jax.ShapeDtypeStruct((N, H), jnp.float32),
    )(x, W_r, b_l)


def _tc_combine(N, D, H, BLK, p0, p1, c0, c1, z, W_l):
    def body(p0_ref, p1_ref, c0_ref, c1_ref, z_ref, wl_ref, o_ref):
        ssum = p0_ref[...] + p1_ref[...]
        cnt = c0_ref[...] + c1_ref[...]
        denom = jnp.maximum(cnt[:, 0:1], 1.0)
        mean = ssum / denom
        o_ref[...] = (
            jnp.dot(mean, wl_ref[...], preferred_element_type=jnp.float32)
            + z_ref[...]
        )

    grid = (N // BLK,)
    return pl.pallas_call(
        body,
        grid=grid,
        in_specs=[
            pl.BlockSpec((BLK, D), lambda i: (i, 0)),
            pl.BlockSpec((BLK, D), lambda i: (i, 0)),
            pl.BlockSpec((BLK, D), lambda i: (i, 0)),
            pl.BlockSpec((BLK, D), lambda i: (i, 0)),
            pl.BlockSpec((BLK, H), lambda i: (i, 0)),
            pl.BlockSpec((D, H), lambda i: (0, 0)),
        ],
        out_specs=pl.BlockSpec((BLK, H), lambda i: (i, 0)),
        out_shape=jax.ShapeDtypeStruct((N, H), jnp.float32),
    )(p0, p1, c0, c1, z, W_l)


def kernel(x, edge_index, W_l, b_l, W_r):
    N, D = x.shape
    E = edge_index.shape[1]
    H = W_l.shape[1]

    src = edge_index[0]
    dst = edge_index[1]

    z = _tc_self(N, D, H, 1000, x, W_r, b_l.reshape(1, H))
    psum = _sc_segment_sum(N, E, D, x, src, dst)
    pcnt = _sc_segment_count(N, E, D, dst)
    out = _tc_combine(N, D, H, 1000, psum[0], psum[1], pcnt[0], pcnt[1], z, W_l)
    return out


# speedup vs baseline: 7.3460x; 1.0126x over previous
import functools

import jax
import jax.numpy as jnp
from jax import lax
from jax.experimental import pallas as pl
from jax.experimental.pallas import tpu as pltpu
from jax.experimental.pallas import tpu_sc as plsc

NC = 2
NS = 16
L = 16

CHUNK = 80
NBUF = 4


def _own_blocks(s, NBLK, ROUNDS, do_copies):
    def body(rnd, carry):
        blk = rnd * NS + s

        @pl.when(blk < NBLK)
        def _():
            do_copies(blk * CHUNK)
        return carry

    lax.fori_loop(0, ROUNDS, body, 0)


def _sc_segment_sum_count(N, E, D, x, src, dst):
    EPW = E // (NC * NS)
    NCHUNK = EPW // CHUNK
    NBLK = N // CHUNK
    ROUNDS = (NBLK + NS - 1) // NS

    mesh = plsc.VectorSubcoreMesh(core_axis_name="c", subcore_axis_name="s")

    @functools.partial(
        pl.kernel,
        out_type=(
            jax.ShapeDtypeStruct((NC, N, D), jnp.float32),
            jax.ShapeDtypeStruct((NC, N, D), jnp.float32),
        ),
        mesh=mesh,
        scratch_types=[
            [pltpu.VMEM((CHUNK,), jnp.int32)] * NBUF,
            [pltpu.VMEM((CHUNK,), jnp.int32)] * NBUF,
            [pltpu.VMEM((CHUNK, D), jnp.float32)] * NBUF,
            pltpu.VMEM_SHARED((N, D), jnp.float32),
            [pltpu.SemaphoreType.DMA] * NBUF,
        ],
    )
    def seg_sum_cnt(x_hbm, src_hbm, dst_hbm, psum_hbm, pcnt_hbm,
                    src_vs, dst_vs, rows_vs, acc_sh, sems):
        c = lax.axis_index("c")
        s = lax.axis_index("s")
        zeros16 = jnp.zeros((L,), jnp.float32)
        ones16 = jnp.ones((L,), jnp.float32)

        def zrow(r, carry):
            for j in range(D // L):
                rows_vs[0][r, pl.ds(j * L, L)] = zeros16
            return carry

        lax.fori_loop(0, CHUNK, zrow, 0)

        _own_blocks(s, NBLK, ROUNDS,
                    lambda r0: pltpu.sync_copy(rows_vs[0], acc_sh.at[pl.ds(r0, CHUNK)]))

        plsc.subcore_barrier()

        base = (c * NS + s) * EPW
        NGRP = NCHUNK // NBUF

        for b in range(NBUF):
            pltpu.sync_copy(src_hbm.at[pl.ds(base + b * CHUNK, CHUNK)], src_vs[b])
            pltpu.async_copy(x_hbm.at[src_vs[b]], rows_vs[b], sems[b])

        def grp_fn(g, carry):
            for b in range(NBUF):
                i = g * NBUF + b
                off = base + i * CHUNK
                pltpu.sync_copy(dst_hbm.at[pl.ds(off, CHUNK)], dst_vs[b])
                pltpu.make_async_copy(x_hbm.at[src_vs[b]], rows_vs[b], sems[b]).wait()
                pltpu.sync_copy(rows_vs[b], acc_sh.at[dst_vs[b]], add=True)

                @pl.when(i + NBUF < NCHUNK)
                def _():
                    offn = base + (i + NBUF) * CHUNK
                    pltpu.sync_copy(src_hbm.at[pl.ds(offn, CHUNK)], src_vs[b])
                    pltpu.async_copy(x_hbm.at[src_vs[b]], rows_vs[b], sems[b])
            return carry

        lax.fori_loop(0, NGRP, grp_fn, 0)

        for i in range(NGRP * NBUF, NCHUNK):
            b = i % NBUF
            off = base + i * CHUNK
            pltpu.sync_copy(dst_hbm.at[pl.ds(off, CHUNK)], dst_vs[b])
            pltpu.make_async_copy(x_hbm.at[src_vs[b]], rows_vs[b], sems[b]).wait()
            pltpu.sync_copy(rows_vs[b], acc_sh.at[dst_vs[b]], add=True)

        plsc.subcore_barrier()

        _own_blocks(s, NBLK, ROUNDS,
                    lambda r0: pltpu.sync_copy(acc_sh.at[pl.ds(r0, CHUNK)],
                                               psum_hbm.at[c, pl.ds(r0, CHUNK)]))

        plsc.subcore_barrier()

        def frow(r, carry):
            for j in range(D // L):
                rows_vs[0][r, pl.ds(j * L, L)] = zeros16
                rows_vs[1][r, pl.ds(j * L, L)] = ones16
            return carry

        lax.fori_loop(0, CHUNK, frow, 0)

        _own_blocks(s, NBLK, ROUNDS,
                    lambda r0: pltpu.sync_copy(rows_vs[0], acc_sh.at[pl.ds(r0, CHUNK)]))

        plsc.subcore_barrier()

        def cgrp_fn(g, carry):
            for b in range(NBUF):
                off = base + (g * NBUF + b) * CHUNK
                pltpu.sync_copy(dst_hbm.at[pl.ds(off, CHUNK)], dst_vs[b])
                pltpu.async_copy(rows_vs[1], acc_sh.at[dst_vs[b]], sems[b], add=True)
            for b in range(NBUF):
                pltpu.make_async_copy(rows_vs[1], acc_sh.at[dst_vs[b]], sems[b]).wait()
            return carry

        lax.fori_loop(0, NGRP, cgrp_fn, 0)

        for i in range(NGRP * NBUF, NCHUNK):
            off = base + i * CHUNK
            pltpu.sync_copy(dst_hbm.at[pl.ds(off, CHUNK)], dst_vs[0])
            pltpu.sync_copy(rows_vs[1], acc_sh.at[dst_vs[0]], add=True)

        plsc.subcore_barrier()

        _own_blocks(s, NBLK, ROUNDS,
                    lambda r0: pltpu.sync_copy(acc_sh.at[pl.ds(r0, CHUNK)],
                                               pcnt_hbm.at[c, pl.ds(r0, CHUNK)]))

    return seg_sum_cnt(x, src, dst)


def _tc_combine(N, D, H, BLK, p0, p1, c0, c1, x, W_l, W_r, b_l):
    def body(p0_ref, p1_ref, c0_ref, c1_ref, x_ref, wl_ref, wr_ref, b_ref, o_ref):
        ssum = p0_ref[...] + p1_ref[...]
        cnt = c0_ref[...] + c1_ref[...]
        denom = jnp.maximum(cnt[:, 0:1], 1.0)
        mean = ssum / denom
        o_ref[...] = (
            jnp.dot(mean, wl_ref[...], preferred_element_type=jnp.float32)
            + jnp.dot(x_ref[...], wr_ref[...], preferred_element_type=jnp.float32)
            + b_ref[...]
        )

    grid = (N // BLK,)
    return pl.pallas_call(
        body,
        grid=grid,
        in_specs=[
            pl.BlockSpec((BLK, D), lambda i: (i, 0)),
            pl.BlockSpec((BLK, D), lambda i: (i, 0)),
            pl.BlockSpec((BLK, D), lambda i: (i, 0)),
            pl.BlockSpec((BLK, D), lambda i: (i, 0)),
            pl.BlockSpec((BLK, D), lambda i: (i, 0)),
            pl.BlockSpec((D, H), lambda i: (0, 0)),
            pl.BlockSpec((D, H), lambda i: (0, 0)),
            pl.BlockSpec((1, H), lambda i: (0, 0)),
        ],
        out_specs=pl.BlockSpec((BLK, H), lambda i: (i, 0)),
        out_shape=jax.ShapeDtypeStruct((N, H), jnp.float32),
    )(p0, p1, c0, c1, x, W_l, W_r, b_l)


def kernel(x, edge_index, W_l, b_l, W_r):
    N, D = x.shape
    E = edge_index.shape[1]
    H = W_l.shape[1]

    src = edge_index[0]
    dst = edge_index[1]

    psum, pcnt = _sc_segment_sum_count(N, E, D, x, src, dst)
    out = _tc_combine(N, D, H, 1000, psum[0], psum[1], pcnt[0], pcnt[1],
                      x, W_l, W_r, b_l.reshape(1, H))
    return out
